# baseline (speedup 1.0000x reference)
import jax
import jax.numpy as jnp
from jax import lax
from jax.experimental import pallas as pl
from jax.experimental.pallas import tpu as pltpu
from jax.experimental.pallas import tpu_sc as plsc

BATCH = 10000
D = 128
DEG = 16
NREL = 3
NC, NS, L = 2, 16, 16
NW = NC * NS
CORE0_N = 448
CORE1_N = 192
NMAX = max(CORE0_N, CORE1_N)
NPAD = NS * (CORE0_N + CORE1_N)
NPAD_X = NPAD + (NMAX - min(CORE0_N, CORE1_N))
CH = 8
NBUF = 4
NVR = D // L
SCH = 16


def _sc_body(alpha_hbm, nodes_hbm, neigh_hbm, feat_hbm,
             self_out, wagg_out,
             alpha_v, w_v, nodes_v, neigh_v, wagg_v,
             st0, st1, st2, st3, sf0, sf1, sem0, sem1, sem2, sem3):
    sts = (st0, st1, st2, st3)
    sfs = (sf0, sf1)
    sems = (sem0, sem1, sem2, sem3)
    cc = lax.axis_index("c")
    ss = lax.axis_index("s")
    nw = jnp.where(cc == 0, CORE0_N, CORE1_N)
    base = jnp.where(cc == 0, ss * CORE0_N, NS * CORE0_N + ss * CORE1_N)
    nch_r = nw // CH
    ntot = NREL * nch_r

    pltpu.sync_copy(nodes_hbm.at[pl.ds(base, NMAX)], nodes_v)
    pltpu.sync_copy(neigh_hbm.at[pl.ds(base * (NREL * DEG), NMAX * NREL * DEG)],
                    neigh_v)

    def self_body(g, carry):
        off0 = g * (2 * SCH)
        cp0 = pltpu.async_copy(
            feat_hbm.at[nodes_v.at[pl.ds(off0, SCH)]], sf0, sem0)
        cp1 = pltpu.async_copy(
            feat_hbm.at[nodes_v.at[pl.ds(off0 + SCH, SCH)]], sf1, sem1)
        cp0.wait()
        pltpu.sync_copy(sf0, self_out.at[pl.ds(base + off0, SCH)])
        cp1.wait()
        pltpu.sync_copy(sf1, self_out.at[pl.ds(base + off0 + SCH, SCH)])
        return carry
    lax.fori_loop(0, nw // (2 * SCH), self_body, 0)

    pltpu.sync_copy(alpha_hbm, alpha_v)
    for i in range(NVR):
        sl = pl.ds(i * L, L)
        a0, a1, a2 = alpha_v[0, sl], alpha_v[1, sl], alpha_v[2, sl]
        m = jnp.maximum(jnp.maximum(a0, a1), a2)
        e0, e1, e2 = jnp.exp(a0 - m), jnp.exp(a1 - m), jnp.exp(a2 - m)
        inv = (1.0 / DEG) / (e0 + e1 + e2)
        w_v[0, sl] = e0 * inv
        w_v[1, sl] = e1 * inv
        w_v[2, sl] = e2 * inv

    zero = jnp.zeros((L,), jnp.float32)

    def zbody(i, c):
        for k in range(NVR):
            wagg_v[0, pl.ds(k * L, L)] = zero
        return c
    lax.fori_loop(0, 1, zbody, 0)

    def fire(c, b):
        off = pl.multiple_of(c * (CH * DEG), CH * DEG)
        return pltpu.async_copy(
            feat_hbm.at[neigh_v.at[pl.ds(off, CH * DEG)]], sts[b], sems[b])

    def process(c, st):
        sl = pl.ds(0, L)
        wagg_v[0, sl] = wagg_v[0, sl] + st[0, sl]

    for b in range(NBUF):
        fire(b, b)

    def main_body(p, carry):
        for b in range(NBUF):
            c = p * NBUF + b
            _wait_chunk(feat_hbm, sts[b], sems[b])
            process(c, sts[b])
            fire(jnp.where(c + NBUF >= ntot, c + NBUF - ntot, c + NBUF), b)
        return carry
    lax.fori_loop(0, ntot // NBUF, main_body, 0)

    for b in range(NBUF):
        _wait_chunk(feat_hbm, sts[b], sems[b])

    def out_body(g, carry):
        off = g * (2 * SCH)
        pltpu.sync_copy(wagg_v.at[pl.ds(0, 2 * SCH)],
                        wagg_out.at[pl.ds(base + off, 2 * SCH)])
        return carry
    lax.fori_loop(0, nw // (2 * SCH), out_body, 0)


def _wait_chunk(feat_hbm, st, sem):
    pltpu.make_async_copy(feat_hbm.at[pl.ds(0, CH * DEG)], st, sem).wait()


def _pad_rows(x, n_rows):
    x = x.astype(jnp.int32)
    pad = n_rows - x.shape[0]
    cfg = [(0, pad)] + [(0, 0)] * (x.ndim - 1)
    return jnp.pad(x, cfg)


def kernel(features, alpha, nodes, neigh1, neigh2, neigh3):
    features = features.astype(jnp.float32)
    alpha_t = alpha[D:, :].T.astype(jnp.float32)
    nodes_p = _pad_rows(nodes, NPAD_X)
    nei = jnp.stack([_pad_rows(n, NPAD) for n in (neigh1, neigh2, neigh3)],
                    axis=1)
    n0 = NS * CORE0_N
    part0 = nei[:n0].reshape(NS, CORE0_N, NREL, DEG)
    part0 = part0.transpose(0, 2, 1, 3).reshape(-1)
    part1 = nei[n0:].reshape(NS, CORE1_N, NREL, DEG)
    part1 = part1.transpose(0, 2, 1, 3).reshape(-1)
    pad = jnp.zeros(((NPAD_X - NPAD) * NREL * DEG,), jnp.int32)
    neigh_p = jnp.concatenate([part0, part1, pad])

    mesh = plsc.VectorSubcoreMesh(core_axis_name="c", subcore_axis_name="s")
    f = pl.kernel(
        _sc_body,
        out_type=(jax.ShapeDtypeStruct((NPAD, D), jnp.float32),
                  jax.ShapeDtypeStruct((NPAD, D), jnp.float32)),
        mesh=mesh,
        scratch_types=(
            pltpu.VMEM((NREL, D), jnp.float32),
            pltpu.VMEM((NREL, D), jnp.float32),
            pltpu.VMEM((NMAX,), jnp.int32),
            pltpu.VMEM((NMAX * NREL * DEG,), jnp.int32),
            pltpu.VMEM((2 * SCH, D), jnp.float32),
            pltpu.VMEM((CH * DEG, D), jnp.float32),
            pltpu.VMEM((CH * DEG, D), jnp.float32),
            pltpu.VMEM((CH * DEG, D), jnp.float32),
            pltpu.VMEM((CH * DEG, D), jnp.float32),
            pltpu.VMEM((SCH, D), jnp.float32),
            pltpu.VMEM((SCH, D), jnp.float32),
            pltpu.SemaphoreType.DMA,
            pltpu.SemaphoreType.DMA,
            pltpu.SemaphoreType.DMA,
            pltpu.SemaphoreType.DMA,
        ),
    )
    self_o, wagg_o = f(alpha_t, nodes_p, neigh_p, features)
    self_o = self_o[:BATCH]
    return jnp.concatenate([self_o, self_o, wagg_o[:BATCH]], axis=1)


# speedup vs baseline: 1.0752x; 1.0752x over previous
import jax
import jax.numpy as jnp
from jax import lax
from jax.experimental import pallas as pl
from jax.experimental.pallas import tpu as pltpu
from jax.experimental.pallas import tpu_sc as plsc

BATCH = 10000
D = 128
DEG = 16
NREL = 3
NC, NS, L = 2, 16, 16
CORE0_N = 448
CORE1_N = 192
NMAX = max(CORE0_N, CORE1_N)
NBIG = 512
NPAD = NS * (CORE0_N + CORE1_N)
NPAD_X = NPAD + NBIG
CH = 8
NBUF = 2
NVR = D // L
OB = 32


def _sc_body(alpha_hbm, nodes_hbm, ne1_hbm, ne2_hbm, ne3_hbm, feat_hbm,
             out_hbm,
             alpha_v, w_v, nodes_v, neigh_v, wagg_v,
             st0, st1, sem0, sem1):
    sts = (st0, st1)
    sems = (sem0, sem1)
    cc = lax.axis_index("c")
    ss = lax.axis_index("s")
    nw = jnp.where(cc == 0, CORE0_N, CORE1_N)
    base = jnp.where(cc == 0, ss * CORE0_N, NS * CORE0_N + ss * CORE1_N)
    nch_r = nw // CH
    ntot = NREL * nch_r

    pltpu.sync_copy(nodes_hbm.at[pl.ds(base, NBIG)], nodes_v)
    for q, ne in enumerate((ne1_hbm, ne2_hbm, ne3_hbm)):
        pltpu.sync_copy(ne.at[pl.ds(base * DEG, NMAX * DEG)],
                        neigh_v.at[pl.ds(q * NMAX * DEG, NMAX * DEG)])

    def sfire(g, carry):
        off = pl.multiple_of(g * (CH * DEG), CH * DEG)
        pltpu.async_copy(feat_hbm.at[nodes_v.at[pl.ds(off, CH * DEG)]],
                         wagg_v.at[pl.ds(off, CH * DEG)], sem0)
        return carry
    ns_ch = (nw + CH * DEG - 1) // (CH * DEG)
    lax.fori_loop(0, ns_ch, sfire, 0)

    pltpu.sync_copy(alpha_hbm, alpha_v)
    for i in range(NVR):
        sl = pl.ds(i * L, L)
        a0, a1, a2 = alpha_v[0, sl], alpha_v[1, sl], alpha_v[2, sl]
        m = jnp.maximum(jnp.maximum(a0, a1), a2)
        e0, e1, e2 = jnp.exp(a0 - m), jnp.exp(a1 - m), jnp.exp(a2 - m)
        inv = (1.0 / DEG) / (e0 + e1 + e2)
        w_v[0, sl] = e0 * inv
        w_v[1, sl] = e1 * inv
        w_v[2, sl] = e2 * inv

    def sdrain(g, carry):
        _wait_self(feat_hbm, wagg_v, sem0)
        return carry
    lax.fori_loop(0, ns_ch, sdrain, 0)

    def self_out_body(g, carry):
        off = g * OB
        blk = wagg_v.at[pl.ds(off, OB)]
        pltpu.sync_copy(blk, out_hbm.at[pl.ds(base + off, OB), pl.ds(0, D)])
        pltpu.sync_copy(blk, out_hbm.at[pl.ds(base + off, OB), pl.ds(D, D)])
        return carry
    lax.fori_loop(0, nw // OB, self_out_body, 0)

    zero = jnp.zeros((L,), jnp.float32)

    def zbody(i, c):
        for k in range(NVR):
            wagg_v[i, pl.ds(k * L, L)] = zero
        return c
    lax.fori_loop(0, nw, zbody, 0)

    def fire(c, b):
        r = c // nch_r
        i0 = (c % nch_r) * (CH * DEG)
        off = pl.multiple_of(r * (NMAX * DEG) + i0, CH * DEG)
        return pltpu.async_copy(
            feat_hbm.at[neigh_v.at[pl.ds(off, CH * DEG)]], sts[b], sems[b])

    def process(c, st):
        r = c // nch_r
        wk = tuple(w_v[r, pl.ds(k * L, L)] for k in range(NVR))
        c0 = (c % nch_r) * CH
        for j in range(CH):
            ci = c0 + j
            for k in range(NVR):
                sl = pl.ds(k * L, L)
                vals = [st[j * DEG + t, sl] for t in range(DEG)]
                while len(vals) > 1:
                    vals = [vals[2 * i] + vals[2 * i + 1]
                            for i in range(len(vals) // 2)]
                wagg_v[ci, sl] = wagg_v[ci, sl] + vals[0] * wk[k]

    for b in range(NBUF):
        fire(b, b)

    def main_body(p, carry):
        for b in range(NBUF):
            c = p * NBUF + b
            _wait_chunk(feat_hbm, sts[b], sems[b])
            process(c, sts[b])
            fire(jnp.where(c + NBUF >= ntot, c + NBUF - ntot, c + NBUF), b)
        return carry
    lax.fori_loop(0, ntot // NBUF, main_body, 0)

    for b in range(NBUF):
        _wait_chunk(feat_hbm, sts[b], sems[b])

    def out_body(g, carry):
        off = g * OB
        pltpu.sync_copy(wagg_v.at[pl.ds(off, OB)],
                        out_hbm.at[pl.ds(base + off, OB), pl.ds(2 * D, D)])
        return carry
    lax.fori_loop(0, nw // OB, out_body, 0)


def _wait_chunk(feat_hbm, st, sem):
    pltpu.make_async_copy(feat_hbm.at[pl.ds(0, CH * DEG)], st, sem).wait()


def _wait_self(feat_hbm, wagg_v, sem):
    pltpu.make_async_copy(feat_hbm.at[pl.ds(0, CH * DEG)],
                          wagg_v.at[pl.ds(0, CH * DEG)], sem).wait()


def _pad_rows(x, n_rows):
    x = x.astype(jnp.int32)
    pad = n_rows - x.shape[0]
    cfg = [(0, pad)] + [(0, 0)] * (x.ndim - 1)
    return jnp.pad(x, cfg)


def kernel(features, alpha, nodes, neigh1, neigh2, neigh3):
    features = features.astype(jnp.float32)
    alpha_t = alpha[D:, :].T.astype(jnp.float32)
    nodes_p = _pad_rows(nodes, NPAD_X)
    ne1 = _pad_rows(neigh1, NPAD_X).reshape(-1)
    ne2 = _pad_rows(neigh2, NPAD_X).reshape(-1)
    ne3 = _pad_rows(neigh3, NPAD_X).reshape(-1)

    mesh = plsc.VectorSubcoreMesh(core_axis_name="c", subcore_axis_name="s")
    f = pl.kernel(
        _sc_body,
        out_type=jax.ShapeDtypeStruct((NPAD, NREL * D), jnp.float32),
        mesh=mesh,
        scratch_types=(
            pltpu.VMEM((NREL, D), jnp.float32),
            pltpu.VMEM((NREL, D), jnp.float32),
            pltpu.VMEM((NBIG,), jnp.int32),
            pltpu.VMEM((NREL * NMAX * DEG,), jnp.int32),
            pltpu.VMEM((NBIG, D), jnp.float32),
            pltpu.VMEM((CH * DEG, D), jnp.float32),
            pltpu.VMEM((CH * DEG, D), jnp.float32),
            pltpu.SemaphoreType.DMA,
            pltpu.SemaphoreType.DMA,
        ),
    )
    out = f(alpha_t, nodes_p, ne1, ne2, ne3, features)
    return out[:BATCH]


# speedup vs baseline: 1.0892x; 1.0130x over previous
import jax
import jax.numpy as jnp
from jax import lax
from jax.experimental import pallas as pl
from jax.experimental.pallas import tpu as pltpu
from jax.experimental.pallas import tpu_sc as plsc

BATCH = 10000
D = 128
DEG = 16
NREL = 3
NC, NS, L = 2, 16, 16
NW = NC * NS
CORE0_N = 448
CORE1_N = 192
NMAX = max(CORE0_N, CORE1_N)
NPAD = NS * (CORE0_N + CORE1_N)
NBIG = 512
NPAD_X = NPAD + NBIG
CH = 8
NBUF = 2
NVR = D // L
SCH = 16


def _sc_body(alpha_hbm, nodes_hbm, neigh_hbm, feat_hbm,
             self_out, wagg_out,
             alpha_v, w_v, nodes_v, neigh_v, wagg_v,
             st0, st1, sem0, sem1):
    sts = (st0, st1)
    sems = (sem0, sem1)
    cc = lax.axis_index("c")
    ss = lax.axis_index("s")
    nw = jnp.where(cc == 0, CORE0_N, CORE1_N)
    base = jnp.where(cc == 0, ss * CORE0_N, NS * CORE0_N + ss * CORE1_N)
    nch_r = nw // CH
    ntot = NREL * nch_r

    pltpu.sync_copy(nodes_hbm.at[pl.ds(base, NBIG)], nodes_v)
    pltpu.sync_copy(neigh_hbm.at[pl.ds(base * (NREL * DEG), NMAX * NREL * DEG)],
                    neigh_v)

    def sfire(g, carry):
        off = pl.multiple_of(g * (CH * DEG), CH * DEG)
        pltpu.async_copy(feat_hbm.at[nodes_v.at[pl.ds(off, CH * DEG)]],
                         wagg_v.at[pl.ds(off, CH * DEG)], sem0)
        return carry
    ns_ch = (nw + CH * DEG - 1) // (CH * DEG)
    lax.fori_loop(0, ns_ch, sfire, 0)

    def sdrain(g, carry):
        pltpu.make_async_copy(feat_hbm.at[pl.ds(0, CH * DEG)],
                              wagg_v.at[pl.ds(0, CH * DEG)], sem0).wait()
        return carry
    lax.fori_loop(0, ns_ch, sdrain, 0)

    def self_out_body(g, carry):
        off = g * (2 * SCH)
        pltpu.sync_copy(wagg_v.at[pl.ds(off, 2 * SCH)],
                        self_out.at[pl.ds(base + off, 2 * SCH)])
        return carry
    lax.fori_loop(0, nw // (2 * SCH), self_out_body, 0)

    pltpu.sync_copy(alpha_hbm, alpha_v)
    for i in range(NVR):
        sl = pl.ds(i * L, L)
        a0, a1, a2 = alpha_v[0, sl], alpha_v[1, sl], alpha_v[2, sl]
        m = jnp.maximum(jnp.maximum(a0, a1), a2)
        e0, e1, e2 = jnp.exp(a0 - m), jnp.exp(a1 - m), jnp.exp(a2 - m)
        inv = (1.0 / DEG) / (e0 + e1 + e2)
        w_v[0, sl] = e0 * inv
        w_v[1, sl] = e1 * inv
        w_v[2, sl] = e2 * inv

    zero = jnp.zeros((L,), jnp.float32)

    def zbody(i, c):
        for k in range(NVR):
            wagg_v[i, pl.ds(k * L, L)] = zero
        return c
    lax.fori_loop(0, nw, zbody, 0)

    def fire(c, b):
        off = pl.multiple_of(c * (CH * DEG), CH * DEG)
        return pltpu.async_copy(
            feat_hbm.at[neigh_v.at[pl.ds(off, CH * DEG)]], sts[b], sems[b])

    def process(c, st):
        r = c // nch_r
        wk = tuple(w_v[r, pl.ds(k * L, L)] for k in range(NVR))
        c0 = (c % nch_r) * CH
        for j in range(CH):
            ci = c0 + j
            for k in range(NVR):
                sl = pl.ds(k * L, L)
                vals = [st[j * DEG + t, sl] for t in range(DEG)]
                while len(vals) > 1:
                    vals = [vals[2 * i] + vals[2 * i + 1]
                            for i in range(len(vals) // 2)]
                wagg_v[ci, sl] = wagg_v[ci, sl] + vals[0] * wk[k]

    for b in range(NBUF):
        fire(b, b)

    def main_body(p, carry):
        for b in range(NBUF):
            c = p * NBUF + b
            _wait_chunk(feat_hbm, sts[b], sems[b])
            process(c, sts[b])
            fire(jnp.where(c + NBUF >= ntot, c + NBUF - ntot, c + NBUF), b)
        return carry
    lax.fori_loop(0, ntot // NBUF, main_body, 0)

    for b in range(NBUF):
        _wait_chunk(feat_hbm, sts[b], sems[b])

    def out_body(g, carry):
        off = g * (2 * SCH)
        pltpu.sync_copy(wagg_v.at[pl.ds(off, 2 * SCH)],
                        wagg_out.at[pl.ds(base + off, 2 * SCH)])
        return carry
    lax.fori_loop(0, nw // (2 * SCH), out_body, 0)


def _wait_chunk(feat_hbm, st, sem):
    pltpu.make_async_copy(feat_hbm.at[pl.ds(0, CH * DEG)], st, sem).wait()


def _pad_rows(x, n_rows):
    x = x.astype(jnp.int32)
    pad = n_rows - x.shape[0]
    cfg = [(0, pad)] + [(0, 0)] * (x.ndim - 1)
    return jnp.pad(x, cfg)


def kernel(features, alpha, nodes, neigh1, neigh2, neigh3):
    features = features.astype(jnp.float32)
    alpha_t = alpha[D:, :].T.astype(jnp.float32)
    nodes_p = _pad_rows(nodes, NPAD_X)
    nei = jnp.stack([_pad_rows(n, NPAD) for n in (neigh1, neigh2, neigh3)],
                    axis=1)
    n0 = NS * CORE0_N
    part0 = nei[:n0].reshape(NS, CORE0_N, NREL, DEG)
    part0 = part0.transpose(0, 2, 1, 3).reshape(-1)
    part1 = nei[n0:].reshape(NS, CORE1_N, NREL, DEG)
    part1 = part1.transpose(0, 2, 1, 3).reshape(-1)
    pad = jnp.zeros(((NPAD_X - NPAD) * NREL * DEG,), jnp.int32)
    neigh_p = jnp.concatenate([part0, part1, pad])

    mesh = plsc.VectorSubcoreMesh(core_axis_name="c", subcore_axis_name="s")
    f = pl.kernel(
        _sc_body,
        out_type=(jax.ShapeDtypeStruct((NPAD, D), jnp.float32),
                  jax.ShapeDtypeStruct((NPAD, D), jnp.float32)),
        mesh=mesh,
        scratch_types=(
            pltpu.VMEM((NREL, D), jnp.float32),
            pltpu.VMEM((NREL, D), jnp.float32),
            pltpu.VMEM((NBIG,), jnp.int32),
            pltpu.VMEM((NMAX * NREL * DEG,), jnp.int32),
            pltpu.VMEM((NBIG, D), jnp.float32),
            pltpu.VMEM((CH * DEG, D), jnp.float32),
            pltpu.VMEM((CH * DEG, D), jnp.float32),
            pltpu.SemaphoreType.DMA,
            pltpu.SemaphoreType.DMA,
        ),
    )
    self_o, wagg_o = f(alpha_t, nodes_p, neigh_p, features)
    self_o = self_o[:BATCH]
    return jnp.concatenate([self_o, self_o, wagg_o[:BATCH]], axis=1)


# speedup vs baseline: 1.0962x; 1.0064x over previous
import jax
import jax.numpy as jnp
from jax import lax
from jax.experimental import pallas as pl
from jax.experimental.pallas import tpu as pltpu
from jax.experimental.pallas import tpu_sc as plsc

BATCH = 10000
D = 128
DEG = 16
NREL = 3
NC, NS, L = 2, 16, 16
NW = NC * NS
CORE0_N = 448
CORE1_N = 192
NMAX = max(CORE0_N, CORE1_N)
NPAD = NS * (CORE0_N + CORE1_N)
NPAD_X = NPAD + (NMAX - min(CORE0_N, CORE1_N))
CH = 8
NBUF = 2
NVR = D // L
SCH = 16


def _sc_body(alpha_hbm, nodes_hbm, neigh_hbm, feat_hbm,
             self_out, wagg_out,
             alpha_v, w_v, nodes_v, neigh_v, wagg_v,
             st0, st1, sf0, sf1, sem0, sem1):
    sts = (st0, st1)
    sfs = (sf0, sf1)
    sems = (sem0, sem1)
    cc = lax.axis_index("c")
    ss = lax.axis_index("s")
    nw = jnp.where(cc == 0, CORE0_N, CORE1_N)
    base = jnp.where(cc == 0, ss * CORE0_N, NS * CORE0_N + ss * CORE1_N)
    nch_r = nw // CH
    ntot = NREL * nch_r

    pltpu.sync_copy(nodes_hbm.at[pl.ds(base, NMAX)], nodes_v)
    pltpu.sync_copy(neigh_hbm.at[pl.ds(base * (NREL * DEG), NMAX * NREL * DEG)],
                    neigh_v)

    def self_body(g, carry):
        off0 = g * (2 * SCH)
        cp0 = pltpu.async_copy(
            feat_hbm.at[nodes_v.at[pl.ds(off0, SCH)]], sf0, sem0)
        cp1 = pltpu.async_copy(
            feat_hbm.at[nodes_v.at[pl.ds(off0 + SCH, SCH)]], sf1, sem1)
        cp0.wait()
        pltpu.sync_copy(sf0, self_out.at[pl.ds(base + off0, SCH)])
        cp1.wait()
        pltpu.sync_copy(sf1, self_out.at[pl.ds(base + off0 + SCH, SCH)])
        return carry
    lax.fori_loop(0, nw // (2 * SCH), self_body, 0)

    pltpu.sync_copy(alpha_hbm, alpha_v)
    for i in range(NVR):
        sl = pl.ds(i * L, L)
        a0, a1, a2 = alpha_v[0, sl], alpha_v[1, sl], alpha_v[2, sl]
        m = jnp.maximum(jnp.maximum(a0, a1), a2)
        e0, e1, e2 = jnp.exp(a0 - m), jnp.exp(a1 - m), jnp.exp(a2 - m)
        inv = (1.0 / DEG) / (e0 + e1 + e2)
        w_v[0, sl] = e0 * inv
        w_v[1, sl] = e1 * inv
        w_v[2, sl] = e2 * inv

    zero = jnp.zeros((L,), jnp.float32)

    def zbody(i, c):
        for k in range(NVR):
            wagg_v[i, pl.ds(k * L, L)] = zero
        return c
    lax.fori_loop(0, nw, zbody, 0)

    def fire(c, b):
        off = pl.multiple_of(c * (CH * DEG), CH * DEG)
        return pltpu.async_copy(
            feat_hbm.at[neigh_v.at[pl.ds(off, CH * DEG)]], sts[b], sems[b])

    def process(c, st):
        r = c // nch_r
        wk = tuple(w_v[r, pl.ds(k * L, L)] for k in range(NVR))
        c0 = (c % nch_r) * CH
        for j in range(CH):
            ci = c0 + j
            for k in range(NVR):
                sl = pl.ds(k * L, L)
                vals = [st[j * DEG + t, sl] for t in range(DEG)]
                while len(vals) > 1:
                    vals = [vals[2 * i] + vals[2 * i + 1]
                            for i in range(len(vals) // 2)]
                wagg_v[ci, sl] = wagg_v[ci, sl] + vals[0] * wk[k]

    for b in range(NBUF):
        fire(b, b)

    def main_body(p, carry):
        for b in range(NBUF):
            c = p * NBUF + b
            _wait_chunk(feat_hbm, sts[b], sems[b])
            process(c, sts[b])
            fire(jnp.where(c + NBUF >= ntot, c + NBUF - ntot, c + NBUF), b)
        return carry
    lax.fori_loop(0, ntot // NBUF, main_body, 0)

    for b in range(NBUF):
        _wait_chunk(feat_hbm, sts[b], sems[b])

    def out_body(g, carry):
        off = g * (2 * SCH)
        pltpu.sync_copy(wagg_v.at[pl.ds(off, 2 * SCH)],
                        wagg_out.at[pl.ds(base + off, 2 * SCH)])
        return carry
    lax.fori_loop(0, nw // (2 * SCH), out_body, 0)


def _wait_chunk(feat_hbm, st, sem):
    pltpu.make_async_copy(feat_hbm.at[pl.ds(0, CH * DEG)], st, sem).wait()


def _pad_rows(x, n_rows):
    x = x.astype(jnp.int32)
    pad = n_rows - x.shape[0]
    cfg = [(0, pad)] + [(0, 0)] * (x.ndim - 1)
    return jnp.pad(x, cfg)


def kernel(features, alpha, nodes, neigh1, neigh2, neigh3):
    features = features.astype(jnp.float32)
    alpha_t = alpha[D:, :].T.astype(jnp.float32)
    nodes_p = _pad_rows(nodes, NPAD_X)
    nei = jnp.stack([_pad_rows(n, NPAD) for n in (neigh1, neigh2, neigh3)],
                    axis=1)
    n0 = NS * CORE0_N
    part0 = nei[:n0].reshape(NS, CORE0_N, NREL, DEG)
    part0 = part0.transpose(0, 2, 1, 3).reshape(-1)
    part1 = nei[n0:].reshape(NS, CORE1_N, NREL, DEG)
    part1 = part1.transpose(0, 2, 1, 3).reshape(-1)
    pad = jnp.zeros(((NPAD_X - NPAD) * NREL * DEG,), jnp.int32)
    neigh_p = jnp.concatenate([part0, part1, pad])

    mesh = plsc.VectorSubcoreMesh(core_axis_name="c", subcore_axis_name="s")
    f = pl.kernel(
        _sc_body,
        out_type=(jax.ShapeDtypeStruct((NPAD, D), jnp.float32),
                  jax.ShapeDtypeStruct((NPAD, D), jnp.float32)),
        mesh=mesh,
        scratch_types=(
            pltpu.VMEM((NREL, D), jnp.float32),
            pltpu.VMEM((NREL, D), jnp.float32),
            pltpu.VMEM((NMAX,), jnp.int32),
            pltpu.VMEM((NMAX * NREL * DEG,), jnp.int32),
            pltpu.VMEM((NMAX, D), jnp.float32),
            pltpu.VMEM((CH * DEG, D), jnp.float32),
            pltpu.VMEM((CH * DEG, D), jnp.float32),
            pltpu.VMEM((SCH, D), jnp.float32),
            pltpu.VMEM((SCH, D), jnp.float32),
            pltpu.SemaphoreType.DMA,
            pltpu.SemaphoreType.DMA,
        ),
    )
    self_o, wagg_o = f(alpha_t, nodes_p, neigh_p, features)
    self_o = self_o[:BATCH]
    return jnp.concatenate([self_o, self_o, wagg_o[:BATCH]], axis=1)


# speedup vs baseline: 1.0990x; 1.0026x over previous
import jax
import jax.numpy as jnp
from jax import lax
from jax.experimental import pallas as pl
from jax.experimental.pallas import tpu as pltpu
from jax.experimental.pallas import tpu_sc as plsc

BATCH = 10000
D = 128
DEG = 16
NREL = 3
NC, NS, L = 2, 16, 16
NW = NC * NS
CORE0_N = 448
CORE1_N = 192
NMAX = max(CORE0_N, CORE1_N)
NPAD = NS * (CORE0_N + CORE1_N)
NPAD_X = NPAD + (NMAX - min(CORE0_N, CORE1_N))
CH = 8
NBUF = 2
NVR = D // L
SCH = 16


def _sc_body(alpha_hbm, nodes_hbm, neigh_hbm, feat_hbm,
             self_out, wagg_out,
             alpha_v, w_v, nodes_v, neigh_v, wagg_v,
             st0, st1, sf0, sf1, sem0, sem1):
    sts = (st0, st1)
    sfs = (sf0, sf1)
    sems = (sem0, sem1)
    cc = lax.axis_index("c")
    ss = lax.axis_index("s")
    nw = jnp.where(cc == 0, CORE0_N, CORE1_N)
    base = jnp.where(cc == 0, ss * CORE0_N, NS * CORE0_N + ss * CORE1_N)
    nch_r = nw // CH
    ntot = NREL * nch_r

    pltpu.sync_copy(nodes_hbm.at[pl.ds(base, NMAX)], nodes_v)
    pltpu.sync_copy(neigh_hbm.at[pl.ds(base * (NREL * DEG), NMAX * NREL * DEG)],
                    neigh_v)

    def self_body(g, carry):
        off0 = g * (2 * SCH)
        cp0 = pltpu.async_copy(
            feat_hbm.at[nodes_v.at[pl.ds(off0, SCH)]], sf0, sem0)
        cp1 = pltpu.async_copy(
            feat_hbm.at[nodes_v.at[pl.ds(off0 + SCH, SCH)]], sf1, sem1)
        cp0.wait()
        pltpu.sync_copy(sf0, self_out.at[pl.ds(base + off0, SCH)])
        cp1.wait()
        pltpu.sync_copy(sf1, self_out.at[pl.ds(base + off0 + SCH, SCH)])
        return carry
    lax.fori_loop(0, nw // (2 * SCH), self_body, 0)

    pltpu.sync_copy(alpha_hbm, alpha_v)
    for i in range(NVR):
        sl = pl.ds(i * L, L)
        a0, a1, a2 = alpha_v[0, sl], alpha_v[1, sl], alpha_v[2, sl]
        m = jnp.maximum(jnp.maximum(a0, a1), a2)
        e0, e1, e2 = jnp.exp(a0 - m), jnp.exp(a1 - m), jnp.exp(a2 - m)
        inv = (1.0 / DEG) / (e0 + e1 + e2)
        w_v[0, sl] = e0 * inv
        w_v[1, sl] = e1 * inv
        w_v[2, sl] = e2 * inv

    zero = jnp.zeros((L,), jnp.float32)

    def zbody(i, c):
        for k in range(NVR):
            wagg_v[i, pl.ds(k * L, L)] = zero
        return c
    lax.fori_loop(0, nw, zbody, 0)

    def fire(c, b):
        off = pl.multiple_of(c * (CH * DEG), CH * DEG)
        for h in range(CH):
            idx = neigh_v[pl.ds(off + h * DEG, DEG)]
            pltpu.async_copy(feat_hbm.at[idx],
                             sts[b].at[pl.ds(h * DEG, DEG)], sems[b])

    def process(c, st):
        r = c // nch_r
        wk = tuple(w_v[r, pl.ds(k * L, L)] for k in range(NVR))
        c0 = (c % nch_r) * CH
        for j in range(CH):
            ci = c0 + j
            for k in range(NVR):
                sl = pl.ds(k * L, L)
                vals = [st[j * DEG + t, sl] for t in range(DEG)]
                while len(vals) > 1:
                    vals = [vals[2 * i] + vals[2 * i + 1]
                            for i in range(len(vals) // 2)]
                wagg_v[ci, sl] = wagg_v[ci, sl] + vals[0] * wk[k]

    for b in range(NBUF):
        fire(b, b)

    def main_body(p, carry):
        for b in range(NBUF):
            c = p * NBUF + b
            _wait_chunk(feat_hbm, sts[b], sems[b])
            process(c, sts[b])
            fire(jnp.where(c + NBUF >= ntot, c + NBUF - ntot, c + NBUF), b)
        return carry
    lax.fori_loop(0, ntot // NBUF, main_body, 0)

    for b in range(NBUF):
        _wait_chunk(feat_hbm, sts[b], sems[b])

    def out_body(g, carry):
        off = g * (2 * SCH)
        pltpu.sync_copy(wagg_v.at[pl.ds(off, 2 * SCH)],
                        wagg_out.at[pl.ds(base + off, 2 * SCH)])
        return carry
    lax.fori_loop(0, nw // (2 * SCH), out_body, 0)


def _wait_chunk(feat_hbm, st, sem):
    pltpu.make_async_copy(feat_hbm.at[pl.ds(0, CH * DEG)], st, sem).wait()


def _pad_rows(x, n_rows):
    x = x.astype(jnp.int32)
    pad = n_rows - x.shape[0]
    cfg = [(0, pad)] + [(0, 0)] * (x.ndim - 1)
    return jnp.pad(x, cfg)


def kernel(features, alpha, nodes, neigh1, neigh2, neigh3):
    features = features.astype(jnp.float32)
    alpha_t = alpha[D:, :].T.astype(jnp.float32)
    nodes_p = _pad_rows(nodes, NPAD_X)
    nei = jnp.stack([_pad_rows(n, NPAD) for n in (neigh1, neigh2, neigh3)],
                    axis=1)
    n0 = NS * CORE0_N
    part0 = nei[:n0].reshape(NS, CORE0_N, NREL, DEG)
    part0 = part0.transpose(0, 2, 1, 3).reshape(-1)
    part1 = nei[n0:].reshape(NS, CORE1_N, NREL, DEG)
    part1 = part1.transpose(0, 2, 1, 3).reshape(-1)
    pad = jnp.zeros(((NPAD_X - NPAD) * NREL * DEG,), jnp.int32)
    neigh_p = jnp.concatenate([part0, part1, pad])

    mesh = plsc.VectorSubcoreMesh(core_axis_name="c", subcore_axis_name="s")
    f = pl.kernel(
        _sc_body,
        out_type=(jax.ShapeDtypeStruct((NPAD, D), jnp.float32),
                  jax.ShapeDtypeStruct((NPAD, D), jnp.float32)),
        mesh=mesh,
        scratch_types=(
            pltpu.VMEM((NREL, D), jnp.float32),
            pltpu.VMEM((NREL, D), jnp.float32),
            pltpu.VMEM((NMAX,), jnp.int32),
            pltpu.VMEM((NMAX * NREL * DEG,), jnp.int32),
            pltpu.VMEM((NMAX, D), jnp.float32),
            pltpu.VMEM((CH * DEG, D), jnp.float32),
            pltpu.VMEM((CH * DEG, D), jnp.float32),
            pltpu.VMEM((SCH, D), jnp.float32),
            pltpu.VMEM((SCH, D), jnp.float32),
            pltpu.SemaphoreType.DMA,
            pltpu.SemaphoreType.DMA,
        ),
    )
    self_o, wagg_o = f(alpha_t, nodes_p, neigh_p, features)
    self_o = self_o[:BATCH]
    return jnp.concatenate([self_o, self_o, wagg_o[:BATCH]], axis=1)
